# Initial kernel scaffold; baseline (speedup 1.0000x reference)
import functools

import jax
import jax.numpy as jnp
from jax import lax
from jax.experimental import pallas as pl
from jax.experimental.pallas import tpu as pltpu
from jax.experimental.pallas import tpu_sc as plsc

N_NODES = 10000
N_EDGES = 320000
D = 128

NC = 2
NS = 16
NW = NC * NS
E_PER_TILE = N_EDGES // NW
K = 80
CHUNKS = E_PER_TILE // K
ROWS_PER_TILE = N_NODES // NS
WB = 125
WB_ITERS = ROWS_PER_TILE // WB

_MESH = plsc.VectorSubcoreMesh(core_axis_name="c", subcore_axis_name="s")


def _sc_cnt_body(h_hbm, src_hbm, dst_hbm, z128_hbm, z16_hbm, ones_hbm,
                 sums_hbm, cnts_hbm,
                 acc_sh, cnt_sh, src_v, dst_v, rows_v, ones_v, zbuf,
                 cbuf, sem):
    c = lax.axis_index("c")
    s = lax.axis_index("s")
    wid = c * NS + s

    pltpu.sync_copy(z128_hbm, zbuf)
    pltpu.sync_copy(z16_hbm, cbuf)
    pltpu.sync_copy(ones_hbm, ones_v)
    for r in range(WB_ITERS):
        row = s * ROWS_PER_TILE + r * WB
        pltpu.sync_copy(zbuf, acc_sh.at[pl.ds(row, WB)])
        pltpu.sync_copy(cbuf, cnt_sh.at[pl.ds(row, WB)])
    plsc.subcore_barrier()

    base = wid * E_PER_TILE

    def chunk(j, carry):
        off = pl.multiple_of(base + j * K, 8)
        pltpu.sync_copy(src_hbm.at[pl.ds(off, K)], src_v)
        pltpu.sync_copy(dst_hbm.at[pl.ds(off, K)], dst_v)
        pltpu.async_copy(h_hbm.at[src_v], rows_v, sem).wait()
        pltpu.sync_copy(rows_v, acc_sh.at[dst_v], add=True)
        pltpu.sync_copy(ones_v, cnt_sh.at[dst_v], add=True)
        return carry

    lax.fori_loop(0, CHUNKS, chunk, 0)
    plsc.subcore_barrier()

    for r in range(WB_ITERS):
        row = s * ROWS_PER_TILE + r * WB
        pltpu.sync_copy(acc_sh.at[pl.ds(row, WB)], zbuf)
        pltpu.sync_copy(zbuf, sums_hbm.at[c, pl.ds(row, WB)])
        pltpu.sync_copy(cnt_sh.at[pl.ds(row, WB)], cbuf)
        pltpu.sync_copy(cbuf, cnts_hbm.at[c, pl.ds(row, WB)])


def _sc_body(h_hbm, src_hbm, dst_hbm, z128_hbm, sums_hbm,
             acc_sh, src_v, dst_v, rows_v, zbuf, sem):
    c = lax.axis_index("c")
    s = lax.axis_index("s")
    wid = c * NS + s

    pltpu.sync_copy(z128_hbm, zbuf)
    for r in range(WB_ITERS):
        row = s * ROWS_PER_TILE + r * WB
        pltpu.sync_copy(zbuf, acc_sh.at[pl.ds(row, WB)])
    plsc.subcore_barrier()

    base = wid * E_PER_TILE

    def chunk(j, carry):
        off = pl.multiple_of(base + j * K, 8)
        pltpu.sync_copy(src_hbm.at[pl.ds(off, K)], src_v)
        pltpu.sync_copy(dst_hbm.at[pl.ds(off, K)], dst_v)
        pltpu.async_copy(h_hbm.at[src_v], rows_v, sem).wait()
        pltpu.sync_copy(rows_v, acc_sh.at[dst_v], add=True)
        return carry

    lax.fori_loop(0, CHUNKS, chunk, 0)
    plsc.subcore_barrier()

    for r in range(WB_ITERS):
        row = s * ROWS_PER_TILE + r * WB
        pltpu.sync_copy(acc_sh.at[pl.ds(row, WB)], zbuf)
        pltpu.sync_copy(zbuf, sums_hbm.at[c, pl.ds(row, WB)])


_sc_aggregate_cnt = pl.kernel(
    _sc_cnt_body,
    out_type=(jax.ShapeDtypeStruct((NC, N_NODES, D), jnp.float32),
              jax.ShapeDtypeStruct((NC, N_NODES, 16), jnp.float32)),
    mesh=_MESH,
    scratch_types=[
        pltpu.VMEM_SHARED((N_NODES, D), jnp.float32),
        pltpu.VMEM_SHARED((N_NODES, 16), jnp.float32),
        pltpu.VMEM((K,), jnp.int32),
        pltpu.VMEM((K,), jnp.int32),
        pltpu.VMEM((K, D), jnp.float32),
        pltpu.VMEM((K, 16), jnp.float32),
        pltpu.VMEM((WB, D), jnp.float32),
        pltpu.VMEM((WB, 16), jnp.float32),
        pltpu.SemaphoreType.DMA,
    ],
    name="sage_sc_aggregate_cnt",
)

_sc_aggregate = pl.kernel(
    _sc_body,
    out_type=(jax.ShapeDtypeStruct((NC, N_NODES, D), jnp.float32),),
    mesh=_MESH,
    scratch_types=[
        pltpu.VMEM_SHARED((N_NODES, D), jnp.float32),
        pltpu.VMEM((K,), jnp.int32),
        pltpu.VMEM((K,), jnp.int32),
        pltpu.VMEM((K, D), jnp.float32),
        pltpu.VMEM((WB, D), jnp.float32),
        pltpu.SemaphoreType.DMA,
    ],
    name="sage_sc_aggregate",
)

ROW_BLK = 1000
GRID = N_NODES // ROW_BLK


def _tc_dense_kernel(last, s_ref, c_ref, h_ref, wl_ref, b_ref, wr_ref,
                     o_ref):
    summed = s_ref[0] + s_ref[1]
    cnt = c_ref[0, :, 0:1] + c_ref[1, :, 0:1]
    mean = summed / jnp.maximum(cnt, 1.0)
    h = h_ref[...]
    z = (jnp.dot(mean, wl_ref[...], preferred_element_type=jnp.float32,
                 precision=lax.Precision.HIGHEST)
         + jnp.dot(h, wr_ref[...], preferred_element_type=jnp.float32,
                   precision=lax.Precision.HIGHEST)
         + b_ref[...])
    z = jnp.where(z > 0, z, jnp.expm1(z))
    if last:
        m = jnp.max(z, axis=1, keepdims=True)
        lse = m + jnp.log(jnp.sum(jnp.exp(z - m), axis=1, keepdims=True))
        z = z - lse
    o_ref[...] = z


def _tc_dense(sums, cnts, h, W_l, b, W_r, last):
    return pl.pallas_call(
        functools.partial(_tc_dense_kernel, last),
        grid=(GRID,),
        in_specs=[
            pl.BlockSpec((NC, ROW_BLK, D), lambda i: (0, i, 0)),
            pl.BlockSpec((NC, ROW_BLK, 16), lambda i: (0, i, 0)),
            pl.BlockSpec((ROW_BLK, D), lambda i: (i, 0)),
            pl.BlockSpec((D, D), lambda i: (0, 0)),
            pl.BlockSpec((1, D), lambda i: (0, 0)),
            pl.BlockSpec((D, D), lambda i: (0, 0)),
        ],
        out_specs=pl.BlockSpec((ROW_BLK, D), lambda i: (i, 0)),
        out_shape=jax.ShapeDtypeStruct((N_NODES, D), jnp.float32),
    )(sums, cnts, h, W_l, b, W_r)


def kernel(x, edge_index, W_l1, b1, W_r1, W_l2, b2, W_r2):
    src = edge_index[0]
    dst = edge_index[1]
    z128 = jnp.zeros((WB, D), jnp.float32)
    z16 = jnp.zeros((WB, 16), jnp.float32)
    ones16 = jnp.ones((K, 16), jnp.float32)
    b1r = b1.reshape(1, D)
    b2r = b2.reshape(1, D)

    sums1, cnts = _sc_aggregate_cnt(x, src, dst, z128, z16, ones16)
    h1 = _tc_dense(sums1, cnts, x, W_l1, b1r, W_r1, last=False)
    (sums2,) = _sc_aggregate(h1, src, dst, z128)
    out = _tc_dense(sums2, cnts, h1, W_l2, b2r, W_r2, last=True)
    return out


# speedup vs baseline: 4.6910x; 4.6910x over previous
import functools

import jax
import jax.numpy as jnp
from jax import lax
from jax.experimental import pallas as pl
from jax.experimental.pallas import tpu as pltpu
from jax.experimental.pallas import tpu_sc as plsc

N_NODES = 10000
N_EDGES = 320000
D = 128

NC = 2
NS = 16
NW = NC * NS
E_PER_TILE = N_EDGES // NW
K = 80
CHUNKS = E_PER_TILE // K
ROWS_PER_TILE = 624
TAIL_ROW = NS * ROWS_PER_TILE
TAIL = N_NODES - TAIL_ROW


def _sc_cnt_body(dst_hbm, z128_hbm, ones_hbm, cnts_hbm,
                 cnt_sh, dst_v, ones_v):
    c = lax.axis_index("c")
    s = lax.axis_index("s")
    wid = c * NS + s
    row = s * ROWS_PER_TILE

    pltpu.sync_copy(ones_hbm, ones_v)
    pltpu.sync_copy(z128_hbm.at[pl.ds(row, ROWS_PER_TILE)],
                    cnt_sh.at[pl.ds(row, ROWS_PER_TILE)])

    @pl.when(s == 0)
    def _():
        pltpu.sync_copy(z128_hbm.at[pl.ds(TAIL_ROW, TAIL)],
                        cnt_sh.at[pl.ds(TAIL_ROW, TAIL)])
    plsc.subcore_barrier()

    base = wid * E_PER_TILE

    def chunk(j, carry):
        off = pl.multiple_of(base + j * K, 8)
        pltpu.sync_copy(dst_hbm.at[pl.ds(off, K)], dst_v)
        pltpu.sync_copy(ones_v, cnt_sh.at[dst_v], add=True)
        return carry

    lax.fori_loop(0, CHUNKS, chunk, 0)
    plsc.subcore_barrier()

    pltpu.sync_copy(cnt_sh.at[pl.ds(row, ROWS_PER_TILE)],
                    cnts_hbm.at[c, pl.ds(row, ROWS_PER_TILE)])

    @pl.when(s == 0)
    def _():
        pltpu.sync_copy(cnt_sh.at[pl.ds(TAIL_ROW, TAIL)],
                        cnts_hbm.at[c, pl.ds(TAIL_ROW, TAIL)])


def _sc_agg_body(h_hbm, src_hbm, dst_hbm, z128_hbm, sums_hbm,
                 acc_sh, src_v, dst_v, rows_v, sem):
    c = lax.axis_index("c")
    s = lax.axis_index("s")
    wid = c * NS + s
    row = s * ROWS_PER_TILE

    pltpu.sync_copy(z128_hbm.at[pl.ds(row, ROWS_PER_TILE)],
                    acc_sh.at[pl.ds(row, ROWS_PER_TILE)])

    @pl.when(s == 0)
    def _():
        pltpu.sync_copy(z128_hbm.at[pl.ds(TAIL_ROW, TAIL)],
                        acc_sh.at[pl.ds(TAIL_ROW, TAIL)])
    plsc.subcore_barrier()

    base = wid * E_PER_TILE

    def chunk(j, carry):
        off = pl.multiple_of(base + j * K, 8)
        pltpu.sync_copy(src_hbm.at[pl.ds(off, K)], src_v)
        pltpu.sync_copy(dst_hbm.at[pl.ds(off, K)], dst_v)
        pltpu.async_copy(h_hbm.at[src_v], rows_v, sem).wait()
        pltpu.sync_copy(rows_v, acc_sh.at[dst_v], add=True)
        return carry

    lax.fori_loop(0, CHUNKS, chunk, 0)
    plsc.subcore_barrier()

    pltpu.sync_copy(acc_sh.at[pl.ds(row, ROWS_PER_TILE)],
                    sums_hbm.at[c, pl.ds(row, ROWS_PER_TILE)])

    @pl.when(s == 0)
    def _():
        pltpu.sync_copy(acc_sh.at[pl.ds(TAIL_ROW, TAIL)],
                        sums_hbm.at[c, pl.ds(TAIL_ROW, TAIL)])


@functools.cache
def _sc_kernels():
    mesh = plsc.VectorSubcoreMesh(core_axis_name="c", subcore_axis_name="s",
                                  num_cores=NC, num_subcores=NS)
    cnt_k = pl.kernel(
        _sc_cnt_body,
        out_type=(jax.ShapeDtypeStruct((NC, N_NODES, D), jnp.float32),),
        mesh=mesh,
        scratch_types=[
            pltpu.VMEM_SHARED((N_NODES, D), jnp.float32),
            pltpu.VMEM((K,), jnp.int32),
            pltpu.VMEM((K, D), jnp.float32),
        ],
        name="sage_sc_count",
    )
    agg_k = pl.kernel(
        _sc_agg_body,
        out_type=(jax.ShapeDtypeStruct((NC, N_NODES, D), jnp.float32),),
        mesh=mesh,
        scratch_types=[
            pltpu.VMEM_SHARED((N_NODES, D), jnp.float32),
            pltpu.VMEM((K,), jnp.int32),
            pltpu.VMEM((K,), jnp.int32),
            pltpu.VMEM((K, D), jnp.float32),
            pltpu.SemaphoreType.DMA,
        ],
        name="sage_sc_aggregate",
    )
    return cnt_k, agg_k


ROW_BLK = 1000
GRID = N_NODES // ROW_BLK


def _tc_dense_kernel(last, s_ref, c_ref, h_ref, wl_ref, b_ref, wr_ref,
                     o_ref):
    summed = s_ref[0] + s_ref[1]
    cnt = c_ref[0, :, 0:1] + c_ref[1, :, 0:1]
    mean = summed / jnp.maximum(cnt, 1.0)
    h = h_ref[...]
    z = (jnp.dot(mean, wl_ref[...], preferred_element_type=jnp.float32,
                 precision=lax.Precision.HIGHEST)
         + jnp.dot(h, wr_ref[...], preferred_element_type=jnp.float32,
                   precision=lax.Precision.HIGHEST)
         + b_ref[...])
    z = jnp.where(z > 0, z, jnp.exp(jnp.minimum(z, 0.0)) - 1.0)
    if last:
        m = jnp.max(z, axis=1, keepdims=True)
        lse = m + jnp.log(jnp.sum(jnp.exp(z - m), axis=1, keepdims=True))
        z = z - lse
    o_ref[...] = z


def _tc_dense(sums, cnts, h, W_l, b, W_r, last):
    return pl.pallas_call(
        functools.partial(_tc_dense_kernel, last),
        grid=(GRID,),
        in_specs=[
            pl.BlockSpec((NC, ROW_BLK, D), lambda i: (0, i, 0)),
            pl.BlockSpec((NC, ROW_BLK, D), lambda i: (0, i, 0)),
            pl.BlockSpec((ROW_BLK, D), lambda i: (i, 0)),
            pl.BlockSpec((D, D), lambda i: (0, 0)),
            pl.BlockSpec((1, D), lambda i: (0, 0)),
            pl.BlockSpec((D, D), lambda i: (0, 0)),
        ],
        out_specs=pl.BlockSpec((ROW_BLK, D), lambda i: (i, 0)),
        out_shape=jax.ShapeDtypeStruct((N_NODES, D), jnp.float32),
    )(sums, cnts, h, W_l, b, W_r)


def kernel(x, edge_index, W_l1, b1, W_r1, W_l2, b2, W_r2):
    src = edge_index[0]
    dst = edge_index[1]
    z128 = jnp.zeros((N_NODES, D), jnp.float32)
    ones128 = jnp.ones((K, D), jnp.float32)
    b1r = b1.reshape(1, D)
    b2r = b2.reshape(1, D)

    cnt_k, agg_k = _sc_kernels()
    (cnts,) = cnt_k(dst, z128, ones128)
    (sums1,) = agg_k(x, src, dst, z128)
    h1 = _tc_dense(sums1, cnts, x, W_l1, b1r, W_r1, last=False)
    (sums2,) = agg_k(h1, src, dst, z128)
    out = _tc_dense(sums2, cnts, h1, W_l2, b2r, W_r2, last=True)
    return out


# speedup vs baseline: 8.4640x; 1.8043x over previous
import functools

import jax
import jax.numpy as jnp
from jax import lax
from jax.experimental import pallas as pl
from jax.experimental.pallas import tpu as pltpu
from jax.experimental.pallas import tpu_sc as plsc

N_NODES = 10000
N_EDGES = 320000
D = 128

NC = 2
NS = 16
NW = NC * NS
E_PER_TILE = N_EDGES // NW
K = 100
CHUNKS = E_PER_TILE // K
HALVES = 2
HCHUNKS = CHUNKS // HALVES
ROWS_PER_TILE = 624
TAIL_ROW = NS * ROWS_PER_TILE
TAIL = N_NODES - TAIL_ROW


def _sc_cnt_body(dst4_hbm, z128_hbm, ones_hbm, cnts_hbm,
                 cnt_sh, dst_all, ones_v, sem0, sem1):
    c = lax.axis_index("c")
    s = lax.axis_index("s")
    wid = c * NS + s
    row = s * ROWS_PER_TILE

    pltpu.sync_copy(ones_hbm, ones_v)
    pltpu.sync_copy(z128_hbm.at[pl.ds(row, ROWS_PER_TILE)],
                    cnt_sh.at[pl.ds(row, ROWS_PER_TILE)])

    @pl.when(s == 0)
    def _():
        pltpu.sync_copy(z128_hbm.at[pl.ds(TAIL_ROW, TAIL)],
                        cnt_sh.at[pl.ds(TAIL_ROW, TAIL)])
    plsc.subcore_barrier()

    for half in range(HALVES):
        pltpu.sync_copy(dst4_hbm.at[wid, half], dst_all)
        pltpu.async_copy(ones_v, cnt_sh.at[dst_all.at[0]], sem0, add=True)
        pltpu.async_copy(ones_v, cnt_sh.at[dst_all.at[1]], sem1, add=True)

        def body(g, carry):
            j = g * 2
            pltpu.make_async_copy(ones_v, cnt_sh.at[dst_all.at[j]],
                                  sem0).wait()
            pltpu.async_copy(ones_v, cnt_sh.at[dst_all.at[j + 2]], sem0,
                             add=True)
            pltpu.make_async_copy(ones_v, cnt_sh.at[dst_all.at[j + 1]],
                                  sem1).wait()
            pltpu.async_copy(ones_v, cnt_sh.at[dst_all.at[j + 3]], sem1,
                             add=True)
            return carry

        lax.fori_loop(0, HCHUNKS // 2 - 1, body, 0)
        pltpu.make_async_copy(ones_v, cnt_sh.at[dst_all.at[HCHUNKS - 2]],
                              sem0).wait()
        pltpu.make_async_copy(ones_v, cnt_sh.at[dst_all.at[HCHUNKS - 1]],
                              sem1).wait()
    plsc.subcore_barrier()

    pltpu.sync_copy(cnt_sh.at[pl.ds(row, ROWS_PER_TILE)],
                    cnts_hbm.at[c, pl.ds(row, ROWS_PER_TILE)])

    @pl.when(s == 0)
    def _():
        pltpu.sync_copy(cnt_sh.at[pl.ds(TAIL_ROW, TAIL)],
                        cnts_hbm.at[c, pl.ds(TAIL_ROW, TAIL)])


def _sc_agg_body(h_hbm, src4_hbm, dst4_hbm, z128_hbm, sums_hbm,
                 acc_sh, src_all, dst_all, rows0, rows1,
                 sg0, sg1, ss0, ss1):
    c = lax.axis_index("c")
    s = lax.axis_index("s")
    wid = c * NS + s
    row = s * ROWS_PER_TILE

    pltpu.sync_copy(z128_hbm.at[pl.ds(row, ROWS_PER_TILE)],
                    acc_sh.at[pl.ds(row, ROWS_PER_TILE)])

    @pl.when(s == 0)
    def _():
        pltpu.sync_copy(z128_hbm.at[pl.ds(TAIL_ROW, TAIL)],
                        acc_sh.at[pl.ds(TAIL_ROW, TAIL)])
    plsc.subcore_barrier()

    for half in range(HALVES):
        pltpu.sync_copy(src4_hbm.at[wid, half], src_all)
        pltpu.sync_copy(dst4_hbm.at[wid, half], dst_all)

        pltpu.async_copy(h_hbm.at[src_all.at[0]], rows0, sg0)
        pltpu.async_copy(h_hbm.at[src_all.at[1]], rows1, sg1)

        def body(g, carry):
            j = g * 2
            pltpu.make_async_copy(h_hbm.at[src_all.at[j]], rows0,
                                  sg0).wait()
            pltpu.async_copy(rows0, acc_sh.at[dst_all.at[j]], ss0,
                             add=True)
            pltpu.make_async_copy(h_hbm.at[src_all.at[j + 1]], rows1,
                                  sg1).wait()
            pltpu.async_copy(rows1, acc_sh.at[dst_all.at[j + 1]], ss1,
                             add=True)
            pltpu.make_async_copy(rows0, acc_sh.at[dst_all.at[j]],
                                  ss0).wait()
            pltpu.async_copy(h_hbm.at[src_all.at[j + 2]], rows0, sg0)
            pltpu.make_async_copy(rows1, acc_sh.at[dst_all.at[j + 1]],
                                  ss1).wait()
            pltpu.async_copy(h_hbm.at[src_all.at[j + 3]], rows1, sg1)
            return carry

        lax.fori_loop(0, HCHUNKS // 2 - 1, body, 0)

        j = HCHUNKS - 2
        pltpu.make_async_copy(h_hbm.at[src_all.at[j]], rows0, sg0).wait()
        pltpu.async_copy(rows0, acc_sh.at[dst_all.at[j]], ss0, add=True)
        pltpu.make_async_copy(h_hbm.at[src_all.at[j + 1]], rows1,
                              sg1).wait()
        pltpu.async_copy(rows1, acc_sh.at[dst_all.at[j + 1]], ss1,
                         add=True)
        pltpu.make_async_copy(rows0, acc_sh.at[dst_all.at[j]],
                              ss0).wait()
        pltpu.make_async_copy(rows1, acc_sh.at[dst_all.at[j + 1]],
                              ss1).wait()
    plsc.subcore_barrier()

    pltpu.sync_copy(acc_sh.at[pl.ds(row, ROWS_PER_TILE)],
                    sums_hbm.at[c, pl.ds(row, ROWS_PER_TILE)])

    @pl.when(s == 0)
    def _():
        pltpu.sync_copy(acc_sh.at[pl.ds(TAIL_ROW, TAIL)],
                        sums_hbm.at[c, pl.ds(TAIL_ROW, TAIL)])


@functools.cache
def _sc_kernels():
    mesh = plsc.VectorSubcoreMesh(core_axis_name="c", subcore_axis_name="s",
                                  num_cores=NC, num_subcores=NS)
    cnt_k = pl.kernel(
        _sc_cnt_body,
        out_type=(jax.ShapeDtypeStruct((NC, N_NODES, D), jnp.float32),),
        mesh=mesh,
        scratch_types=[
            pltpu.VMEM_SHARED((N_NODES, D), jnp.float32),
            pltpu.VMEM((HCHUNKS, K), jnp.int32),
            pltpu.VMEM((K, D), jnp.float32),
            pltpu.SemaphoreType.DMA,
            pltpu.SemaphoreType.DMA,
        ],
        name="sage_sc_count",
    )
    agg_k = pl.kernel(
        _sc_agg_body,
        out_type=(jax.ShapeDtypeStruct((NC, N_NODES, D), jnp.float32),),
        mesh=mesh,
        scratch_types=[
            pltpu.VMEM_SHARED((N_NODES, D), jnp.float32),
            pltpu.VMEM((HCHUNKS, K), jnp.int32),
            pltpu.VMEM((HCHUNKS, K), jnp.int32),
            pltpu.VMEM((K, D), jnp.float32),
            pltpu.VMEM((K, D), jnp.float32),
            pltpu.SemaphoreType.DMA,
            pltpu.SemaphoreType.DMA,
            pltpu.SemaphoreType.DMA,
            pltpu.SemaphoreType.DMA,
        ],
        name="sage_sc_aggregate",
    )
    return cnt_k, agg_k


ROW_BLK = 1000
GRID = N_NODES // ROW_BLK


def _tc_dense_kernel(last, s_ref, c_ref, h_ref, wl_ref, b_ref, wr_ref,
                     o_ref):
    summed = s_ref[0] + s_ref[1]
    cnt = c_ref[0, :, 0:1] + c_ref[1, :, 0:1]
    mean = summed / jnp.maximum(cnt, 1.0)
    h = h_ref[...]
    z = (jnp.dot(mean, wl_ref[...], preferred_element_type=jnp.float32,
                 precision=lax.Precision.HIGHEST)
         + jnp.dot(h, wr_ref[...], preferred_element_type=jnp.float32,
                   precision=lax.Precision.HIGHEST)
         + b_ref[...])
    z = jnp.where(z > 0, z, jnp.exp(jnp.minimum(z, 0.0)) - 1.0)
    if last:
        m = jnp.max(z, axis=1, keepdims=True)
        lse = m + jnp.log(jnp.sum(jnp.exp(z - m), axis=1, keepdims=True))
        z = z - lse
    o_ref[...] = z


def _tc_dense(sums, cnts, h, W_l, b, W_r, last):
    return pl.pallas_call(
        functools.partial(_tc_dense_kernel, last),
        grid=(GRID,),
        in_specs=[
            pl.BlockSpec((NC, ROW_BLK, D), lambda i: (0, i, 0)),
            pl.BlockSpec((NC, ROW_BLK, D), lambda i: (0, i, 0)),
            pl.BlockSpec((ROW_BLK, D), lambda i: (i, 0)),
            pl.BlockSpec((D, D), lambda i: (0, 0)),
            pl.BlockSpec((1, D), lambda i: (0, 0)),
            pl.BlockSpec((D, D), lambda i: (0, 0)),
        ],
        out_specs=pl.BlockSpec((ROW_BLK, D), lambda i: (i, 0)),
        out_shape=jax.ShapeDtypeStruct((N_NODES, D), jnp.float32),
    )(sums, cnts, h, W_l, b, W_r)


def kernel(x, edge_index, W_l1, b1, W_r1, W_l2, b2, W_r2):
    src4 = edge_index[0].reshape(NW, HALVES, HCHUNKS, K)
    dst4 = edge_index[1].reshape(NW, HALVES, HCHUNKS, K)
    z128 = jnp.zeros((N_NODES, D), jnp.float32)
    ones128 = jnp.ones((K, D), jnp.float32)
    b1r = b1.reshape(1, D)
    b2r = b2.reshape(1, D)

    cnt_k, agg_k = _sc_kernels()
    (cnts,) = cnt_k(dst4, z128, ones128)
    (sums1,) = agg_k(x, src4, dst4, z128)
    h1 = _tc_dense(sums1, cnts, x, W_l1, b1r, W_r1, last=False)
    (sums2,) = agg_k(h1, src4, dst4, z128)
    out = _tc_dense(sums2, cnts, h1, W_l2, b2r, W_r2, last=True)
    return out
